# baseline (speedup 1.0000x reference)
import functools

import jax
import jax.numpy as jnp
from jax.experimental import pallas as pl
from jax.experimental.pallas import tpu as pltpu

_K = 128
_D = 32
_L = 8
_B = 4096
_N = _B * _L
_ROWS = 2048
_GRID = _N // _ROWS
_BB = _ROWS // _L


def _body(x_ref, cb_ref, cbf_ref, pol_ref, qnt_ref, cset_ref):
    x = x_ref[...]
    cb = cb_ref[...]
    prod = jax.lax.dot_general(
        x, cb, (((1,), (1,)), ((), ())),
        preferred_element_type=jnp.float32)
    dist = (jnp.sum(x * x, axis=1, keepdims=True)
            + jnp.sum(cb * cb, axis=1)[None, :]) - 2.0 * prod
    mins = jnp.min(dist, axis=1, keepdims=True)
    iota = jax.lax.broadcasted_iota(jnp.int32, dist.shape, 1)
    idx = jnp.min(jnp.where(dist == mins, iota, _K), axis=1, keepdims=True)
    onehot = (iota == idx).astype(jnp.float32)
    q = jax.lax.dot_general(
        onehot, cb, (((1,), (0,)), ((), ())),
        preferred_element_type=jnp.float32)
    pol_ref[...] = q
    qnt_ref[...] = q
    cset_ref[...] = jnp.broadcast_to(cbf_ref[...], cset_ref.shape)


@functools.partial(jax.jit, static_argnames=())
def kernel(latent, codebook):
    flat = latent.reshape(_N, _D)
    cb_flat = codebook.reshape(1, _K * _D)
    pol, qnt, cset = pl.pallas_call(
        _body,
        grid=(_GRID,),
        in_specs=[
            pl.BlockSpec((_ROWS, _D), lambda i: (i, 0)),
            pl.BlockSpec((_K, _D), lambda i: (0, 0)),
            pl.BlockSpec((1, _K * _D), lambda i: (0, 0)),
        ],
        out_specs=[
            pl.BlockSpec((_ROWS, _D), lambda i: (i, 0)),
            pl.BlockSpec((_ROWS, _D), lambda i: (i, 0)),
            pl.BlockSpec((_BB, _K * _D), lambda i: (i, 0)),
        ],
        out_shape=[
            jax.ShapeDtypeStruct((_N, _D), jnp.float32),
            jax.ShapeDtypeStruct((_N, _D), jnp.float32),
            jax.ShapeDtypeStruct((_B, _K * _D), jnp.float32),
        ],
        compiler_params=pltpu.CompilerParams(
            dimension_semantics=("arbitrary",),
        ),
    )(flat, codebook, cb_flat)
    shape = latent.shape
    return (pol.reshape(shape), qnt.reshape(shape),
            cset.reshape(_B, _K, _D))
